# baseline (speedup 1.0000x reference)
import jax
import jax.numpy as jnp
from jax import lax
from jax.experimental import pallas as pl
from jax.experimental.pallas import tpu as pltpu
from jax.experimental.pallas import tpu_sc as plsc

N = 16384
L = 50
B_DIM = 16
FEAT = 16
OUT_W = B_DIM + FEAT

NC = 2
NS = 16
NW = NC * NS
LANE = 128
NT = N // LANE
TPW = NT // NW
CH = OUT_W // 8
CHF = FEAT // 8
NP = 2


def _transpose_rows(rows2d, kbuf_tp):
    base = lax.iota(jnp.int32, 16)
    for k in range(LANE // 16):
        nl_idx = base + (16 * k)
        for c in range(B_DIM):
            c_idx = jnp.full((16,), c, jnp.int32)
            val = plsc.load_gather(rows2d, [nl_idx, c_idx])
            kbuf_tp[c // 8, c % 8, pl.ds(16 * k, 16)] = val


def _sc_body(idx_hbm, feat_hbm, table_hbm, out_hbm,
             idx_v, rows_v, kbuf, *sems):
    NSL = TPW * NP
    gsem = sems[0:NSL]
    fsem = sems[NSL:2 * NSL]
    wsem = sems[2 * NSL:3 * NSL]
    wid = lax.axis_index("s") * NC + lax.axis_index("c")
    nt0 = wid * TPW

    pltpu.sync_copy(idx_hbm.at[:, pl.ds(nt0, TPW)], idx_v)

    def issue_gather(l, t, p):
        pltpu.async_copy(
            table_hbm.at[idx_v.at[l, t]], rows_v.at[t, p], gsem[t * NP + p])

    def drain_gather(l, t, p):
        pltpu.make_async_copy(
            table_hbm.at[idx_v.at[l, t]], rows_v.at[t, p],
            gsem[t * NP + p]).wait()

    for p in range(NP):
        for t in range(TPW):
            issue_gather(p, t, p)

    def round_body(r, _):
        for p in range(NP):
            l = NP * r + p
            for t in range(TPW):
                nt = nt0 + t
                s = t * NP + p

                @pl.when(l >= NP)
                def _():
                    for ch in range(CH):
                        pltpu.make_async_copy(
                            kbuf.at[t, p, ch], out_hbm.at[l, ch, nt],
                            wsem[s]).wait()

                for chf in range(CHF):
                    pltpu.async_copy(
                        feat_hbm.at[l, chf, nt],
                        kbuf.at[t, p, B_DIM // 8 + chf], fsem[s])
                drain_gather(l, t, p)
                _transpose_rows(rows_v.at[t, p], kbuf.at[t, p])

                @pl.when(l + NP < L)
                def _():
                    issue_gather(l + NP, t, p)

                for chf in range(CHF):
                    pltpu.make_async_copy(
                        feat_hbm.at[l, chf, nt],
                        kbuf.at[t, p, B_DIM // 8 + chf], fsem[s]).wait()
                for ch in range(CH):
                    pltpu.async_copy(
                        kbuf.at[t, p, ch], out_hbm.at[l, ch, nt], wsem[s])
        return ()

    lax.fori_loop(0, L // NP, round_body, (), unroll=False)
    for p in range(NP):
        for t in range(TPW):
            for ch in range(CH):
                pltpu.make_async_copy(
                    kbuf.at[t, p, ch], out_hbm.at[L - NP + p, ch, nt0 + t],
                    wsem[t * NP + p]).wait()


@jax.jit
def _run(idx3, feat5, table):
    kern = pl.kernel(
        _sc_body,
        out_type=jax.ShapeDtypeStruct((L, CH, NT, 8, LANE), jnp.float32),
        mesh=plsc.VectorSubcoreMesh(
            core_axis_name="c", subcore_axis_name="s",
            num_cores=NC, num_subcores=NS,
        ),
        scratch_types=[
            pltpu.VMEM((L, TPW, LANE), jnp.int32),
            pltpu.VMEM((TPW, NP, LANE, B_DIM), jnp.float32),
            pltpu.VMEM((TPW, NP, CH, 8, LANE), jnp.float32),
        ] + [pltpu.SemaphoreType.DMA] * (3 * TPW * NP),
        compiler_params=pltpu.CompilerParams(
            use_tc_tiling_on_sc=False, needs_layout_passes=False),
    )
    return kern(idx3, feat5, table)


def kernel(segmentId, pathSegmentFeat, table):
    idx3 = segmentId.astype(jnp.int32).reshape(N, L).T.reshape(L, NT, LANE)
    feat5 = (pathSegmentFeat.transpose(1, 2, 0)
             .reshape(L, CHF, 8, NT, LANE).transpose(0, 1, 3, 2, 4))
    out = _run(idx3, feat5, table)
    return out.transpose(2, 4, 0, 1, 3).reshape(N, L, OUT_W)


# speedup vs baseline: 1.0014x; 1.0014x over previous
import jax
import jax.numpy as jnp
from jax import lax
from jax.experimental import pallas as pl
from jax.experimental.pallas import tpu as pltpu
from jax.experimental.pallas import tpu_sc as plsc

N = 16384
L = 50
B_DIM = 16
FEAT = 16
OUT_W = B_DIM + FEAT

NC = 2
NS = 16
NW = NC * NS
LANE = 128
NT = N // LANE
TPW = NT // NW
CH = OUT_W // 8
CHF = FEAT // 8
NP = 2


def _transpose_rows(rows2d, kbuf_t):
    base = lax.iota(jnp.int32, 16)
    for k in range(LANE // 16):
        nl_idx = base + (16 * k)
        for c in range(B_DIM):
            c_idx = jnp.full((16,), c, jnp.int32)
            val = plsc.load_gather(rows2d, [nl_idx, c_idx])
            kbuf_t[c // 8, c % 8, pl.ds(16 * k, 16)] = val


def _sc_body(idx_hbm, feat_hbm, table_hbm, out_hbm,
             idx_v, rows_v, kbuf, *sems):
    NSL = TPW * NP
    gsem = sems[0:NSL]
    fsem = sems[NSL:NSL + TPW]
    wsem = sems[NSL + TPW:NSL + 2 * TPW]
    wid = lax.axis_index("s") * NC + lax.axis_index("c")
    nt0 = wid * TPW

    pltpu.sync_copy(idx_hbm.at[:, pl.ds(nt0, TPW)], idx_v)

    def issue_gather(l, t, p):
        pltpu.async_copy(
            table_hbm.at[idx_v.at[l, t]], rows_v.at[t, p], gsem[t * NP + p])

    def drain_gather(l, t, p):
        pltpu.make_async_copy(
            table_hbm.at[idx_v.at[l, t]], rows_v.at[t, p],
            gsem[t * NP + p]).wait()

    for p in range(NP):
        for t in range(TPW):
            issue_gather(p, t, p)

    def round_body(r, _):
        for p in range(NP):
            l = NP * r + p
            for t in range(TPW):
                nt = nt0 + t

                @pl.when(l > 0)
                def _():
                    for ch in range(CH):
                        pltpu.make_async_copy(
                            kbuf.at[t, ch], out_hbm.at[l, ch, nt],
                            wsem[t]).wait()

                for chf in range(CHF):
                    pltpu.async_copy(
                        feat_hbm.at[l, chf, nt],
                        kbuf.at[t, B_DIM // 8 + chf], fsem[t])
                drain_gather(l, t, p)
                _transpose_rows(rows_v.at[t, p], kbuf.at[t])

                @pl.when(l + NP < L)
                def _():
                    issue_gather(l + NP, t, p)

                for chf in range(CHF):
                    pltpu.make_async_copy(
                        feat_hbm.at[l, chf, nt],
                        kbuf.at[t, B_DIM // 8 + chf], fsem[t]).wait()
                for ch in range(CH):
                    pltpu.async_copy(
                        kbuf.at[t, ch], out_hbm.at[l, ch, nt], wsem[t])
        return ()

    lax.fori_loop(0, L // NP, round_body, (), unroll=False)
    for t in range(TPW):
        for ch in range(CH):
            pltpu.make_async_copy(
                kbuf.at[t, ch], out_hbm.at[L - 1, ch, nt0 + t],
                wsem[t]).wait()


@jax.jit
def _run(idx3, feat5, table):
    kern = pl.kernel(
        _sc_body,
        out_type=jax.ShapeDtypeStruct((L, CH, NT, 8, LANE), jnp.float32),
        mesh=plsc.VectorSubcoreMesh(
            core_axis_name="c", subcore_axis_name="s",
            num_cores=NC, num_subcores=NS,
        ),
        scratch_types=[
            pltpu.VMEM((L, TPW, LANE), jnp.int32),
            pltpu.VMEM((TPW, NP, LANE, B_DIM), jnp.float32),
            pltpu.VMEM((TPW, CH, 8, LANE), jnp.float32),
        ] + [pltpu.SemaphoreType.DMA] * (TPW * NP + 2 * TPW),
        compiler_params=pltpu.CompilerParams(
            use_tc_tiling_on_sc=False, needs_layout_passes=False),
    )
    return kern(idx3, feat5, table)


def kernel(segmentId, pathSegmentFeat, table):
    idx3 = segmentId.astype(jnp.int32).reshape(N, L).T.reshape(L, NT, LANE)
    feat5 = (pathSegmentFeat.transpose(1, 2, 0)
             .reshape(L, CHF, 8, NT, LANE).transpose(0, 1, 3, 2, 4))
    out = _run(idx3, feat5, table)
    return out.transpose(2, 4, 0, 1, 3).reshape(N, L, OUT_W)


# speedup vs baseline: 1.0173x; 1.0159x over previous
import jax
import jax.numpy as jnp
from jax import lax
from jax.experimental import pallas as pl
from jax.experimental.pallas import tpu as pltpu
from jax.experimental.pallas import tpu_sc as plsc

N = 16384
L = 50
B_DIM = 16
FEAT = 16
OUT_W = B_DIM + FEAT

NC = 2
NS = 16
NW = NC * NS
LANE = 128
NT = N // LANE
TPW = NT // NW
CH = OUT_W // 8
CHF = FEAT // 8


def _transpose_rows(rows2d, kbuf_t):
    base = lax.iota(jnp.int32, 16)
    for k in range(LANE // 16):
        nl_idx = base + (16 * k)
        for c in range(B_DIM):
            c_idx = jnp.full((16,), c, jnp.int32)
            val = plsc.load_gather(rows2d, [nl_idx, c_idx])
            kbuf_t[c // 8, c % 8, pl.ds(16 * k, 16)] = val


def _sc_body(idx_hbm, feat_hbm, table_hbm, out_hbm,
             idx_v, rows_v, kbuf, *sems):
    gsem = sems[0:TPW]
    fsem = sems[TPW:2 * TPW]
    wsem = sems[2 * TPW:3 * TPW]
    wid = lax.axis_index("s") * NC + lax.axis_index("c")
    nt0 = wid * TPW

    pltpu.sync_copy(idx_hbm.at[:, pl.ds(nt0, TPW)], idx_v)

    def issue_gather(l, t):
        pltpu.async_copy(
            table_hbm.at[idx_v.at[l, t]], rows_v.at[t], gsem[t])

    def drain_gather(l, t):
        pltpu.make_async_copy(
            table_hbm.at[idx_v.at[l, t]], rows_v.at[t], gsem[t]).wait()

    for t in range(TPW):
        issue_gather(0, t)

    def l_body(l, _):
        for t in range(TPW):
            nt = nt0 + t

            @pl.when(l > 0)
            def _():
                for ch in range(CH):
                    pltpu.make_async_copy(
                        kbuf.at[t, ch], out_hbm.at[l, ch, nt], wsem[t]
                    ).wait()

            drain_gather(l, t)
            for chf in range(CHF):
                pltpu.async_copy(
                    feat_hbm.at[l, chf, nt], kbuf.at[t, B_DIM // 8 + chf],
                    fsem[t])
            _transpose_rows(rows_v.at[t], kbuf.at[t])

            @pl.when(l + 1 < L)
            def _():
                issue_gather(l + 1, t)

            for chf in range(CHF):
                pltpu.make_async_copy(
                    feat_hbm.at[l, chf, nt], kbuf.at[t, B_DIM // 8 + chf],
                    fsem[t]).wait()
            for ch in range(CH):
                pltpu.async_copy(
                    kbuf.at[t, ch], out_hbm.at[l, ch, nt], wsem[t])
        return ()

    lax.fori_loop(0, L, l_body, (), unroll=False)
    for t in range(TPW):
        for ch in range(CH):
            pltpu.make_async_copy(
                kbuf.at[t, ch], out_hbm.at[L - 1, ch, nt0 + t], wsem[t]
            ).wait()


@jax.jit
def _run(idx3, feat5, table):
    kern = pl.kernel(
        _sc_body,
        out_type=jax.ShapeDtypeStruct((L, CH, NT, 8, LANE), jnp.float32),
        mesh=plsc.VectorSubcoreMesh(
            core_axis_name="c", subcore_axis_name="s",
            num_cores=NC, num_subcores=NS,
        ),
        scratch_types=[
            pltpu.VMEM((L, TPW, LANE), jnp.int32),
            pltpu.VMEM((TPW, LANE, B_DIM), jnp.float32),
            pltpu.VMEM((TPW, CH, 8, LANE), jnp.float32),
        ] + [pltpu.SemaphoreType.DMA] * (3 * TPW),
        compiler_params=pltpu.CompilerParams(
            use_tc_tiling_on_sc=False, needs_layout_passes=False),
    )
    return kern(idx3, feat5, table)


def kernel(segmentId, pathSegmentFeat, table):
    idx3 = segmentId.astype(jnp.int32).reshape(N, L).T.reshape(L, NT, LANE)
    feat5 = (pathSegmentFeat.transpose(1, 2, 0)
             .reshape(L, CHF, 8, NT, LANE).transpose(0, 1, 3, 2, 4))
    out = _run(idx3, feat5, table)
    return out.transpose(2, 4, 0, 1, 3).reshape(N, L, OUT_W)


# speedup vs baseline: 1.0182x; 1.0009x over previous
import jax
import jax.numpy as jnp
from jax import lax
from jax.experimental import pallas as pl
from jax.experimental.pallas import tpu as pltpu
from jax.experimental.pallas import tpu_sc as plsc

N = 16384
L = 50
B_DIM = 16
FEAT = 16
OUT_W = B_DIM + FEAT

NC = 2
NS = 16
NW = NC * NS
LANE = 128
NT = N // LANE
TPW = NT // NW
CH = OUT_W // 8
CHF = FEAT // 8


def _transpose_rows(rows2d, kbuf_t):
    base = lax.iota(jnp.int32, 16)
    for k in range(LANE // 16):
        nl_idx = base + (16 * k)
        for c in range(B_DIM):
            c_idx = jnp.full((16,), c, jnp.int32)
            val = plsc.load_gather(rows2d, [nl_idx, c_idx])
            kbuf_t[c // 8, c % 8, pl.ds(16 * k, 16)] = val


def _sc_body(idx_hbm, feat_hbm, table_hbm, out_hbm,
             idx_v, rows_v, kbuf, *sems):
    gsem = sems[0:TPW]
    fsem = sems[TPW:2 * TPW]
    wsem = sems[2 * TPW:3 * TPW]
    wid = lax.axis_index("s") * NC + lax.axis_index("c")
    nt0 = wid * TPW

    pltpu.sync_copy(idx_hbm.at[:, pl.ds(nt0, TPW)], idx_v)

    def issue_gather(l, t):
        pltpu.async_copy(
            table_hbm.at[idx_v.at[l, t]], rows_v.at[t], gsem[t])

    def drain_gather(l, t):
        pltpu.make_async_copy(
            table_hbm.at[idx_v.at[l, t]], rows_v.at[t], gsem[t]).wait()

    for t in range(TPW):
        issue_gather(0, t)

    def l_body(l, _):
        for t in range(TPW):
            nt = nt0 + t

            @pl.when(l > 0)
            def _():
                for ch in range(CH):
                    pltpu.make_async_copy(
                        kbuf.at[t, ch], out_hbm.at[l, ch, nt], wsem[t]
                    ).wait()

            drain_gather(l, t)
            for chf in range(CHF):
                pltpu.async_copy(
                    feat_hbm.at[l, chf, nt], kbuf.at[t, B_DIM // 8 + chf],
                    fsem[t])
            _transpose_rows(rows_v.at[t], kbuf.at[t])

            @pl.when(l + 1 < L)
            def _():
                issue_gather(l + 1, t)

            for chf in range(CHF):
                pltpu.make_async_copy(
                    feat_hbm.at[l, chf, nt], kbuf.at[t, B_DIM // 8 + chf],
                    fsem[t]).wait()
            for ch in range(CH):
                pltpu.async_copy(
                    kbuf.at[t, ch], out_hbm.at[l, ch, nt], wsem[t])
        return ()

    lax.fori_loop(0, L, l_body, (), unroll=False)
    for t in range(TPW):
        for ch in range(CH):
            pltpu.make_async_copy(
                kbuf.at[t, ch], out_hbm.at[L - 1, ch, nt0 + t], wsem[t]
            ).wait()


@jax.jit
def _run(idx3, feat5, table):
    kern = pl.kernel(
        _sc_body,
        out_type=jax.ShapeDtypeStruct((L, CH, NT, 8, LANE), jnp.float32),
        mesh=plsc.VectorSubcoreMesh(
            core_axis_name="c", subcore_axis_name="s",
            num_cores=NC, num_subcores=NS,
        ),
        scratch_types=[
            pltpu.VMEM((L, TPW, LANE), jnp.int32),
            pltpu.VMEM((TPW, LANE, B_DIM), jnp.float32),
            pltpu.VMEM((TPW, CH, 8, LANE), jnp.float32),
        ] + [pltpu.SemaphoreType.DMA] * (3 * TPW),
        compiler_params=pltpu.CompilerParams(
            use_tc_tiling_on_sc=False, needs_layout_passes=False),
    )
    return kern(idx3, feat5, table)


def kernel(segmentId, pathSegmentFeat, table):
    idx3 = segmentId.astype(jnp.int32).reshape(N, L).T.reshape(L, NT, LANE)
    feat5 = (pathSegmentFeat.transpose(1, 2, 0)
             .reshape(L, CHF, 8, NT, LANE).transpose(0, 1, 3, 2, 4))
    tflat = lax.optimization_barrier(table.reshape(-1))
    table_rm = tflat.reshape(table.shape)
    out = _run(idx3, feat5, table_rm)
    return out.transpose(2, 4, 0, 1, 3).reshape(N, L, OUT_W)
